# baseline (speedup 1.0000x reference)
import jax
import jax.numpy as jnp
from jax import lax
from jax.experimental import pallas as pl
from jax.experimental.pallas import tpu as pltpu
from jax.experimental.pallas import tpu_sc as plsc

B = 4
N = 4096
K = 16
G = 4
CQ = 64
CV = 64
CQG = CQ // G
CVG = CV // G
DTBL = 144
DOWN = 68
NC = 2
NS = 16
PTS_PER_TILE = (B * N) // (NC * NS)
CHUNK = 16
NCHUNK = PTS_PER_TILE // CHUNK
MCOL = N // 8


def _rcp(x):
    i = plsc.bitcast(x, jnp.int32)
    i = jnp.int32(0x7EF311C3) - i
    y = plsc.bitcast(i, jnp.float32)
    for _ in range(3):
        y = y * (2.0 - x * y)
    return y


def _rsqrt(s):
    i = plsc.bitcast(s, jnp.int32)
    i = jnp.int32(0x5F3759DF) - (i >> 1)
    y = plsc.bitcast(i, jnp.float32)
    for _ in range(3):
        y = y * (1.5 - 0.5 * s * y * y)
    return y


def _sc_body(qtbl, vtbl, xtbl, nidx_hbm, base_hbm, params_hbm,
             lv_hbm, fx_hbm, cent_hbm, cpart_hbm,
             nbqA, nbvA, nbxA, nbqB, nbvB, nbxB,
             gidxA, gidxB, oqA, oxA, oqB, oxB, fxA, fxB, lvA, lvB,
             nidx_all, la_buf, cent_buf, params_v, base_v,
             macc, mbuf,
             semgA, semgB, semoA, semoB, semwA, semwB):
    c_idx = lax.axis_index("c")
    s_idx = lax.axis_index("s")
    bl = s_idx // 8
    b = 2 * c_idx + bl
    part = s_idx % 8
    n0 = part * PTS_PER_TILE
    p0 = b * N + n0

    iota = lax.iota(jnp.int32, 16)

    def gather_descs(nbq, nbv, nbx, gidx, semg):
        ds = []
        for h in range(2):
            sl = pl.ds(h * 128, 128)
            ds.append(pltpu.make_async_copy(
                qtbl.at[gidx.at[h]], nbq.at[sl, :], semg))
            ds.append(pltpu.make_async_copy(
                vtbl.at[gidx.at[h]], nbv.at[sl, :], semg))
            ds.append(pltpu.make_async_copy(
                xtbl.at[gidx.at[h]], nbx.at[sl, :], semg))
        return ds

    def own_descs(oq, ox, chunk, semo):
        pb = p0 + chunk * CHUNK
        return (
            pltpu.make_async_copy(qtbl.at[pl.ds(pb, CHUNK), :], oq, semo),
            pltpu.make_async_copy(xtbl.at[pl.ds(pb, CHUNK), :], ox, semo),
        )

    def write_descs(fx, lv, chunk, semw):
        nc = n0 + chunk * CHUNK
        return [
            pltpu.make_async_copy(
                fx, fx_hbm.at[b, :, :, pl.ds(nc, CHUNK)], semw),
            pltpu.make_async_copy(
                lv, lv_hbm.at[b, :, pl.ds(nc, CHUNK)], semw),
        ]

    def build_gidx(gidx, chunk, bsv):
        for i in range(CHUNK):
            v = nidx_all[chunk * CHUNK + i, :] + bsv
            gidx[i // 8, pl.ds((i % 8) * 16, 16)] = v

    pltpu.sync_copy(params_hbm, params_v)
    pltpu.sync_copy(base_hbm, base_v)
    pltpu.sync_copy(nidx_hbm.at[pl.ds(p0, PTS_PER_TILE), :], nidx_all)
    bsv = plsc.load_gather(base_v, [jnp.full((16,), b, jnp.int32)])

    zero16 = jnp.zeros((16,), jnp.float32)

    def zero_body(i, _):
        for g in range(G):
            cent_buf[g, pl.ds(i * 16, 16)] = zero16
        return 0
    lax.fori_loop(0, N // 16, zero_body, 0)

    build_gidx(gidxA, 0, bsv)
    for d in gather_descs(nbqA, nbvA, nbxA, gidxA, semgA):
        d.start()
    for d in own_descs(oqA, oxA, 0, semoA):
        d.start()
    build_gidx(gidxB, 1, bsv)
    for d in gather_descs(nbqB, nbvB, nbxB, gidxB, semgB):
        d.start()
    for d in own_descs(oqB, oxB, 1, semoB):
        d.start()

    wdv = params_v[0, :]
    axv = params_v[1, :]
    ayv = params_v[2, :]
    azv = params_v[3, :]
    bxv = params_v[4, :]
    byv = params_v[5, :]
    bzv = params_v[6, :]
    bbv = params_v[7, :]
    im1 = jnp.maximum(iota - 1, 0)
    ip1 = jnp.minimum(iota + 1, 15)

    c16 = [jnp.full((16,), c, jnp.int32) for c in range(DTBL)]

    def splat(ref, idxs):
        return plsc.load_gather(ref, idxs)

    def lane_take(x, idx):
        return x.at[idx].get(mode="promise_in_bounds")

    def compute_chunk(nbq, nbv, nbx, oqr, oxr, fx, lv, chunk):
        @plsc.parallel_loop(0, CHUNK, 1, unroll=2)
        def point(pp):
            r0 = pp * 16
            rows = r0 + iota
            ppv = jnp.full((16,), pp, jnp.int32)
            x0 = splat(oxr, [ppv, c16[0]])
            y0 = splat(oxr, [ppv, c16[1]])
            z0 = splat(oxr, [ppv, c16[2]])

            las = []
            for g in range(G):
                oqv = oqr[pp, pl.ds(g * CQG, CQG)]
                acc0 = zero16
                acc1 = zero16
                for cc in range(CQG):
                    col = g * CQG + cc
                    lk = plsc.load_gather(nbq, [rows, c16[col]])
                    oq = lane_take(oqv, c16[cc])
                    if cc % 2 == 0:
                        acc0 = acc0 + oq * lk
                    else:
                        acc1 = acc1 + oq * lk
                acc = acc0 + acc1
                m = jnp.max(acc)
                e = jnp.exp(acc - m)
                la_g = e / jnp.broadcast_to(jnp.sum(e), (16,))
                la_buf[g, pp, :] = la_g
                las.append(la_g)

            xn = plsc.load_gather(nbx, [rows, c16[0]])
            yn = plsc.load_gather(nbx, [rows, c16[1]])
            zn = plsc.load_gather(nbx, [rows, c16[2]])
            dx = x0 - xn
            dy = y0 - yn
            dz = z0 - zn
            s2 = dx * dx + dy * dy + dz * dz + 1e-12
            dist = s2 * _rsqrt(s2)
            base_vec = x0 * axv + y0 * ayv + z0 * azv + bbv

            accv = [zero16] * G
            accf = [zero16] * G
            for j in range(K):
                fx0j = (base_vec + lane_take(dist, c16[j]) * wdv
                        + lane_take(xn, c16[j]) * bxv
                        + lane_take(yn, c16[j]) * byv
                        + lane_take(zn, c16[j]) * bzv)
                fx0j = jnp.maximum(fx0j, 0.0)
                plsc.store_scatter(fx, [iota, c16[j], ppv], fx0j)
                for g in range(G):
                    laj = lane_take(las[g], c16[j])
                    vrow = nbv[r0 + j, pl.ds(g * CVG, CVG)]
                    accv[g] = accv[g] + laj * vrow
                    accf[g] = accf[g] + laj * fx0j

            for g in range(G):
                plsc.store_scatter(lv, [g * 2 * CVG + iota, ppv], accv[g])
                plsc.store_scatter(lv, [g * 2 * CVG + CVG + iota, ppv],
                                   accf[g])

        @plsc.parallel_loop(0, CHUNK, 1, unroll=2)
        def cent_point(pp):
            nk = nidx_all[chunk * CHUNK + pp, :]
            for g in range(G):
                la_g = la_buf[g, pp, :]
                sk, sv = plsc.sort_key_val(nk, la_g)
                cum = plsc.cumsum(sv)
                prev_k = lane_take(sk, im1)
                next_k = lane_take(sk, ip1)
                prev_c = lane_take(cum, im1)
                first = (iota == 0) | (sk != prev_k)
                last = (iota == 15) | (sk != next_k)
                plsc.addupdate_scatter(cent_buf, [c16[g], sk], cum, mask=last)
                plsc.addupdate_scatter(cent_buf, [c16[g], sk], -prev_c,
                                       mask=first & (iota != 0))

    def half(nbq, nbv, nbx, gidx, oq, ox, fx, lv, semg, semo, semw, chunk):
        for d in gather_descs(nbq, nbv, nbx, gidx, semg):
            d.wait()
        for d in own_descs(oq, ox, chunk, semo):
            d.wait()

        @pl.when(chunk >= 2)
        def _():
            for d in write_descs(fx, lv, chunk, semw):
                d.wait()

        compute_chunk(nbq, nbv, nbx, oq, ox, fx, lv, chunk)

        @pl.when(chunk + 2 < NCHUNK)
        def _():
            build_gidx(gidx, chunk + 2, bsv)
            for d in gather_descs(nbq, nbv, nbx, gidx, semg):
                d.start()
            for d in own_descs(oq, ox, chunk + 2, semo):
                d.start()

        for d in write_descs(fx, lv, chunk, semw):
            d.start()

    def step(t2, _):
        half(nbqA, nbvA, nbxA, gidxA, oqA, oxA, fxA, lvA,
             semgA, semoA, semwA, 2 * t2)
        half(nbqB, nbvB, nbxB, gidxB, oqB, oxB, fxB, lvB,
             semgB, semoB, semwB, 2 * t2 + 1)
        return 0

    lax.fori_loop(0, NCHUNK // 2, step, 0)

    for d in write_descs(fxA, lvA, NCHUNK - 2, semwA):
        d.wait()
    for d in write_descs(fxB, lvB, NCHUNK - 1, semwB):
        d.wait()

    pltpu.sync_copy(cent_buf, cpart_hbm.at[c_idx, s_idx])
    plsc.subcore_barrier()
    blm = s_idx // 8
    colm = (s_idx % 8) * MCOL
    pltpu.sync_copy(cpart_hbm.at[c_idx, blm * 8, :, pl.ds(colm, MCOL)], macc)
    for t in range(1, 8):
        pltpu.sync_copy(cpart_hbm.at[c_idx, blm * 8 + t, :, pl.ds(colm, MCOL)],
                        mbuf)

        def add_body(i, _):
            for g in range(G):
                sl = pl.ds(i * 16, 16)
                macc[g, sl] = macc[g, sl] + mbuf[g, sl]
            return 0
        lax.fori_loop(0, MCOL // 16, add_body, 0)
    pltpu.sync_copy(macc, cent_hbm.at[2 * c_idx + blm, :, pl.ds(colm, MCOL)])


@jax.jit
def kernel(xyz, query, value, neigh_idx, idx_base, W_lpe, bn_gamma, bn_beta,
           bn_mean, bn_var):
    f32 = jnp.float32
    qT = jnp.transpose(query, (0, 2, 1)).reshape(B * N, CQ).astype(f32)
    vT = jnp.transpose(value[..., 0], (0, 2, 1)).reshape(B * N, CV).astype(f32)
    xT = jnp.pad(xyz.reshape(B * N, 3).astype(f32), ((0, 0), (0, 13)))
    nidx = neigh_idx.reshape(B * N, K).astype(jnp.int32)
    base = jnp.zeros((16,), jnp.int32).at[:B].set(
        idx_base.reshape(B).astype(jnp.int32))

    s = bn_gamma / jnp.sqrt(bn_var + 1e-5)
    Wp = W_lpe * s[:, None]
    bp = bn_beta - bn_mean * s
    A = Wp[:, 1:4] + Wp[:, 4:7]
    Bm = Wp[:, 7:10] - Wp[:, 1:4]
    params = jnp.stack([Wp[:, 0], A[:, 0], A[:, 1], A[:, 2],
                        Bm[:, 0], Bm[:, 1], Bm[:, 2], bp]).astype(f32)

    mesh = plsc.VectorSubcoreMesh(core_axis_name="c", subcore_axis_name="s")
    run = pl.kernel(
        _sc_body,
        out_type=(
            jax.ShapeDtypeStruct((B, 2 * CV, N), f32),
            jax.ShapeDtypeStruct((B, CVG, K, N), f32),
            jax.ShapeDtypeStruct((B, G, N), f32),
            jax.ShapeDtypeStruct((NC, NS, G, N), f32),
        ),
        mesh=mesh,
        compiler_params=pltpu.CompilerParams(
            needs_layout_passes=False, use_tc_tiling_on_sc=False),
        scratch_types=[
            pltpu.VMEM((256, CQ), f32),
            pltpu.VMEM((256, CV), f32),
            pltpu.VMEM((256, 16), f32),
            pltpu.VMEM((256, CQ), f32),
            pltpu.VMEM((256, CV), f32),
            pltpu.VMEM((256, 16), f32),
            pltpu.VMEM((2, 128), jnp.int32),
            pltpu.VMEM((2, 128), jnp.int32),
            pltpu.VMEM((CHUNK, CQ), f32),
            pltpu.VMEM((CHUNK, 16), f32),
            pltpu.VMEM((CHUNK, CQ), f32),
            pltpu.VMEM((CHUNK, 16), f32),
            pltpu.VMEM((CVG, K, CHUNK), f32),
            pltpu.VMEM((CVG, K, CHUNK), f32),
            pltpu.VMEM((2 * CV, CHUNK), f32),
            pltpu.VMEM((2 * CV, CHUNK), f32),
            pltpu.VMEM((PTS_PER_TILE, K), jnp.int32),
            pltpu.VMEM((G, CHUNK, K), f32),
            pltpu.VMEM((G, N), f32),
            pltpu.VMEM((8, 16), f32),
            pltpu.VMEM((16,), jnp.int32),
            pltpu.VMEM((G, MCOL), f32),
            pltpu.VMEM((G, MCOL), f32),
            pltpu.SemaphoreType.DMA,
            pltpu.SemaphoreType.DMA,
            pltpu.SemaphoreType.DMA,
            pltpu.SemaphoreType.DMA,
            pltpu.SemaphoreType.DMA,
            pltpu.SemaphoreType.DMA,
        ],
    )
    lv, fx0, cent, _ = run(qT, vT, xT, nidx, base, params)
    fx = jnp.broadcast_to(
        jnp.transpose(fx0, (0, 1, 3, 2))[:, None], (B, G, CVG, N, K))
    return lv[..., None], fx, cent


# speedup vs baseline: 1.2729x; 1.2729x over previous
import jax
import jax.numpy as jnp
from jax import lax
from jax.experimental import pallas as pl
from jax.experimental.pallas import tpu as pltpu
from jax.experimental.pallas import tpu_sc as plsc

B = 4
N = 4096
K = 16
G = 4
CQ = 64
CV = 64
CQG = CQ // G
CVG = CV // G
DTBL = 144
DOWN = 68
NC = 2
NS = 16
PTS_PER_TILE = (B * N) // (NC * NS)
CHUNK = 16
NCHUNK = PTS_PER_TILE // CHUNK
MCOL = N // 8


def _rcp(x):
    i = plsc.bitcast(x, jnp.int32)
    i = jnp.int32(0x7EF311C3) - i
    y = plsc.bitcast(i, jnp.float32)
    for _ in range(3):
        y = y * (2.0 - x * y)
    return y


def _rsqrt(s):
    i = plsc.bitcast(s, jnp.int32)
    i = jnp.int32(0x5F3759DF) - (i >> 1)
    y = plsc.bitcast(i, jnp.float32)
    for _ in range(3):
        y = y * (1.5 - 0.5 * s * y * y)
    return y


def _sc_body(tbl, nidx_hbm, base_hbm, params_hbm,
             lv_hbm, fx_hbm, cent_hbm, cpart_hbm,
             nbA, nbB, gidxA, gidxB, ownA, ownB, fxA, fxB, lvA, lvB,
             nidx_all, la_buf, cent_buf, params_v, base_v,
             macc, mbuf,
             semgA, semgB, semoA, semoB, semwA, semwB):
    c_idx = lax.axis_index("c")
    s_idx = lax.axis_index("s")
    bl = s_idx // 8
    b = 2 * c_idx + bl
    part = s_idx % 8
    n0 = part * PTS_PER_TILE
    p0 = b * N + n0

    iota = lax.iota(jnp.int32, 16)

    def gather_descs(nb, gidx, semg):
        return (pltpu.make_async_copy(tbl.at[gidx.at[0]],
                                      nb.at[pl.ds(0, 128), :], semg),
                pltpu.make_async_copy(tbl.at[gidx.at[1]],
                                      nb.at[pl.ds(128, 128), :], semg))

    def own_desc(own, chunk, semo):
        pb = p0 + chunk * CHUNK
        return pltpu.make_async_copy(tbl.at[pl.ds(pb, CHUNK), :], own, semo)

    def write_descs(fx, lv, chunk, semw):
        nc = n0 + chunk * CHUNK
        return [
            pltpu.make_async_copy(
                fx, fx_hbm.at[b, :, :, pl.ds(nc, CHUNK)], semw),
            pltpu.make_async_copy(
                lv, lv_hbm.at[b, :, pl.ds(nc, CHUNK)], semw),
        ]

    def build_gidx(gidx, chunk, bsv):
        for i in range(CHUNK):
            v = nidx_all[chunk * CHUNK + i, :] + bsv
            gidx[i // 8, pl.ds((i % 8) * 16, 16)] = v

    pltpu.sync_copy(params_hbm, params_v)
    pltpu.sync_copy(base_hbm, base_v)
    pltpu.sync_copy(nidx_hbm.at[pl.ds(p0, PTS_PER_TILE), :], nidx_all)
    bsv = plsc.load_gather(base_v, [jnp.full((16,), b, jnp.int32)])

    zero16 = jnp.zeros((16,), jnp.float32)

    def zero_body(i, _):
        for g in range(G):
            cent_buf[g, pl.ds(i * 16, 16)] = zero16
        return 0
    lax.fori_loop(0, N // 16, zero_body, 0)

    build_gidx(gidxA, 0, bsv)
    for d in gather_descs(nbA, gidxA, semgA):
        d.start()
    own_desc(ownA, 0, semoA).start()
    build_gidx(gidxB, 1, bsv)
    for d in gather_descs(nbB, gidxB, semgB):
        d.start()
    own_desc(ownB, 1, semoB).start()

    wdv = params_v[0, :]
    axv = params_v[1, :]
    ayv = params_v[2, :]
    azv = params_v[3, :]
    bxv = params_v[4, :]
    byv = params_v[5, :]
    bzv = params_v[6, :]
    bbv = params_v[7, :]
    im1 = jnp.maximum(iota - 1, 0)
    ip1 = jnp.minimum(iota + 1, 15)

    c16 = [jnp.full((16,), c, jnp.int32) for c in range(DTBL)]

    def splat(ref, idxs):
        return plsc.load_gather(ref, idxs)

    def lane_take(x, idx):
        return x.at[idx].get(mode="promise_in_bounds")

    def compute_chunk(nb, own, fx, lv, chunk):
        @plsc.parallel_loop(0, CHUNK, 1, unroll=2)
        def point(pp):
            r0 = pp * 16
            rows = r0 + iota
            ppv = jnp.full((16,), pp, jnp.int32)
            x0 = splat(own, [ppv, c16[128]])
            y0 = splat(own, [ppv, c16[129]])
            z0 = splat(own, [ppv, c16[130]])

            las = []
            for g in range(G):
                oqv = own[pp, pl.ds(g * CQG, CQG)]
                acc0 = zero16
                acc1 = zero16
                for cc in range(CQG):
                    col = g * CQG + cc
                    lk = plsc.load_gather(nb, [rows, c16[col]])
                    oq = lane_take(oqv, c16[cc])
                    if cc % 2 == 0:
                        acc0 = acc0 + oq * lk
                    else:
                        acc1 = acc1 + oq * lk
                acc = acc0 + acc1
                m = jnp.max(acc)
                e = jnp.exp(acc - m)
                la_g = e / jnp.broadcast_to(jnp.sum(e), (16,))
                la_buf[g, pp, :] = la_g
                las.append(la_g)

            xn = plsc.load_gather(nb, [rows, c16[128]])
            yn = plsc.load_gather(nb, [rows, c16[129]])
            zn = plsc.load_gather(nb, [rows, c16[130]])
            dx = x0 - xn
            dy = y0 - yn
            dz = z0 - zn
            s2 = dx * dx + dy * dy + dz * dz + 1e-12
            dist = s2 * _rsqrt(s2)
            base_vec = x0 * axv + y0 * ayv + z0 * azv + bbv

            accv = [zero16] * G
            accf = [zero16] * G
            for j in range(K):
                fx0j = (base_vec + lane_take(dist, c16[j]) * wdv
                        + lane_take(xn, c16[j]) * bxv
                        + lane_take(yn, c16[j]) * byv
                        + lane_take(zn, c16[j]) * bzv)
                fx0j = jnp.maximum(fx0j, 0.0)
                plsc.store_scatter(fx, [iota, c16[j], ppv], fx0j)
                for g in range(G):
                    laj = lane_take(las[g], c16[j])
                    vrow = nb[r0 + j, pl.ds(CQ + g * CVG, CVG)]
                    accv[g] = accv[g] + laj * vrow
                    accf[g] = accf[g] + laj * fx0j

            for g in range(G):
                plsc.store_scatter(lv, [g * 2 * CVG + iota, ppv], accv[g])
                plsc.store_scatter(lv, [g * 2 * CVG + CVG + iota, ppv],
                                   accf[g])

        @plsc.parallel_loop(0, CHUNK, 1, unroll=2)
        def cent_point(pp):
            nk = nidx_all[chunk * CHUNK + pp, :]
            for g in range(G):
                la_g = la_buf[g, pp, :]
                sk, sv = plsc.sort_key_val(nk, la_g)
                cum = plsc.cumsum(sv)
                prev_k = lane_take(sk, im1)
                next_k = lane_take(sk, ip1)
                prev_c = lane_take(cum, im1)
                first = (iota == 0) | (sk != prev_k)
                last = (iota == 15) | (sk != next_k)
                plsc.addupdate_scatter(cent_buf, [c16[g], sk], cum, mask=last)
                plsc.addupdate_scatter(cent_buf, [c16[g], sk], -prev_c,
                                       mask=first & (iota != 0))

    def half(nb, gidx, own, fx, lv, semg, semo, semw, chunk):
        for d in gather_descs(nb, gidx, semg):
            d.wait()
        own_desc(own, chunk, semo).wait()

        @pl.when(chunk >= 2)
        def _():
            for d in write_descs(fx, lv, chunk, semw):
                d.wait()

        compute_chunk(nb, own, fx, lv, chunk)

        @pl.when(chunk + 2 < NCHUNK)
        def _():
            build_gidx(gidx, chunk + 2, bsv)
            for d in gather_descs(nb, gidx, semg):
                d.start()
            own_desc(own, chunk + 2, semo).start()

        for d in write_descs(fx, lv, chunk, semw):
            d.start()

    def step(t2, _):
        half(nbA, gidxA, ownA, fxA, lvA, semgA, semoA, semwA, 2 * t2)
        half(nbB, gidxB, ownB, fxB, lvB, semgB, semoB, semwB, 2 * t2 + 1)
        return 0

    lax.fori_loop(0, NCHUNK // 2, step, 0)

    for d in write_descs(fxA, lvA, NCHUNK - 2, semwA):
        d.wait()
    for d in write_descs(fxB, lvB, NCHUNK - 1, semwB):
        d.wait()

    pltpu.sync_copy(cent_buf, cpart_hbm.at[c_idx, s_idx])
    plsc.subcore_barrier()
    blm = s_idx // 8
    colm = (s_idx % 8) * MCOL
    pltpu.sync_copy(cpart_hbm.at[c_idx, blm * 8, :, pl.ds(colm, MCOL)], macc)
    for t in range(1, 8):
        pltpu.sync_copy(cpart_hbm.at[c_idx, blm * 8 + t, :, pl.ds(colm, MCOL)],
                        mbuf)

        def add_body(i, _):
            for g in range(G):
                sl = pl.ds(i * 16, 16)
                macc[g, sl] = macc[g, sl] + mbuf[g, sl]
            return 0
        lax.fori_loop(0, MCOL // 16, add_body, 0)
    pltpu.sync_copy(macc, cent_hbm.at[2 * c_idx + blm, :, pl.ds(colm, MCOL)])


@jax.jit
def kernel(xyz, query, value, neigh_idx, idx_base, W_lpe, bn_gamma, bn_beta,
           bn_mean, bn_var):
    f32 = jnp.float32
    qT = jnp.transpose(query, (0, 2, 1)).reshape(B * N, CQ).astype(f32)
    vT = jnp.transpose(value[..., 0], (0, 2, 1)).reshape(B * N, CV).astype(f32)
    xf = xyz.reshape(B * N, 3).astype(f32)
    zpad = jnp.zeros((B * N, DTBL - CQ - CV - 3), f32)
    tbl = jnp.concatenate([qT, vT, xf, zpad], axis=1)
    nidx = neigh_idx.reshape(B * N, K).astype(jnp.int32)
    base = jnp.zeros((16,), jnp.int32).at[:B].set(
        idx_base.reshape(B).astype(jnp.int32))

    s = bn_gamma / jnp.sqrt(bn_var + 1e-5)
    Wp = W_lpe * s[:, None]
    bp = bn_beta - bn_mean * s
    A = Wp[:, 1:4] + Wp[:, 4:7]
    Bm = Wp[:, 7:10] - Wp[:, 1:4]
    params = jnp.stack([Wp[:, 0], A[:, 0], A[:, 1], A[:, 2],
                        Bm[:, 0], Bm[:, 1], Bm[:, 2], bp]).astype(f32)

    mesh = plsc.VectorSubcoreMesh(core_axis_name="c", subcore_axis_name="s")
    run = pl.kernel(
        _sc_body,
        out_type=(
            jax.ShapeDtypeStruct((B, 2 * CV, N), f32),
            jax.ShapeDtypeStruct((B, CVG, K, N), f32),
            jax.ShapeDtypeStruct((B, G, N), f32),
            jax.ShapeDtypeStruct((NC, NS, G, N), f32),
        ),
        mesh=mesh,
        compiler_params=pltpu.CompilerParams(
            needs_layout_passes=False, use_tc_tiling_on_sc=False),
        scratch_types=[
            pltpu.VMEM((256, DTBL), f32),
            pltpu.VMEM((256, DTBL), f32),
            pltpu.VMEM((2, 128), jnp.int32),
            pltpu.VMEM((2, 128), jnp.int32),
            pltpu.VMEM((CHUNK, DTBL), f32),
            pltpu.VMEM((CHUNK, DTBL), f32),
            pltpu.VMEM((CVG, K, CHUNK), f32),
            pltpu.VMEM((CVG, K, CHUNK), f32),
            pltpu.VMEM((2 * CV, CHUNK), f32),
            pltpu.VMEM((2 * CV, CHUNK), f32),
            pltpu.VMEM((PTS_PER_TILE, K), jnp.int32),
            pltpu.VMEM((G, CHUNK, K), f32),
            pltpu.VMEM((G, N), f32),
            pltpu.VMEM((8, 16), f32),
            pltpu.VMEM((16,), jnp.int32),
            pltpu.VMEM((G, MCOL), f32),
            pltpu.VMEM((G, MCOL), f32),
            pltpu.SemaphoreType.DMA,
            pltpu.SemaphoreType.DMA,
            pltpu.SemaphoreType.DMA,
            pltpu.SemaphoreType.DMA,
            pltpu.SemaphoreType.DMA,
            pltpu.SemaphoreType.DMA,
        ],
    )
    lv, fx0, cent, _ = run(tbl, nidx, base, params)
    fx = jnp.broadcast_to(
        jnp.transpose(fx0, (0, 1, 3, 2))[:, None], (B, G, CVG, N, K))
    return lv[..., None], fx, cent
